# baseline (speedup 1.0000x reference)
import functools

import jax
import jax.numpy as jnp
from jax import lax
from jax.experimental import pallas as pl
from jax.experimental.pallas import tpu as pltpu
from jax.experimental.pallas import tpu_sc as plsc

B = 16384
D = 16
GRP = 8
TROWS = 1000000 // GRP

_NC, _NS = 2, 16
_NW = _NC * _NS
_BPW = B // _NW
_CHUNK = 128
_NB = 4


@functools.cache
def _build_gather4():
    mesh = plsc.VectorSubcoreMesh(core_axis_name="c", subcore_axis_name="s")

    @functools.partial(
        pl.kernel,
        mesh=mesh,
        out_type=[jax.ShapeDtypeStruct((B, GRP * D), jnp.float32)] * 4,
        scratch_types=[
            pltpu.VMEM((_BPW,), jnp.int32),
            pltpu.VMEM((_BPW,), jnp.int32),
        ] + [pltpu.VMEM((_CHUNK, GRP * D), jnp.float32)] * _NB
          + [pltpu.SemaphoreType.DMA] * (2 * _NB),
    )
    def gather4(sid8_hbm, pid8_hbm, esg, epg, esm, epm,
                o_sg, o_pg, o_sm, o_pm,
                sidv, pidv, *scratch):
        bufs = scratch[:_NB]
        gsems = scratch[_NB:2 * _NB]
        osems = scratch[2 * _NB:]
        wid = lax.axis_index("s") * _NC + lax.axis_index("c")
        base = wid * _BPW
        pltpu.sync_copy(sid8_hbm.at[pl.ds(base, _BPW)], sidv)
        pltpu.sync_copy(pid8_hbm.at[pl.ds(base, _BPW)], pidv)

        work = []
        for tab, idx, out in ((esg, sidv, o_sg), (epg, pidv, o_pg),
                              (esm, sidv, o_sm), (epm, pidv, o_pm)):
            for c in range(_BPW // _CHUNK):
                work.append((tab, idx, out, c))
        nwk = len(work)

        g = [None] * nwk
        o = [None] * nwk

        def issue_out(j):
            _, _, out, c = work[j]
            p = j % _NB
            o[j] = pltpu.async_copy(
                bufs[p], out.at[pl.ds(base + c * _CHUNK, _CHUNK)], osems[p])

        for k in range(nwk):
            p = k % _NB
            if k >= _NB:
                o[k - _NB].wait()
            tab, idx, _, c = work[k]
            g[k] = pltpu.async_copy(
                tab.at[idx.at[pl.ds(c * _CHUNK, _CHUNK)]], bufs[p], gsems[p])
            j = k - (_NB - 1)
            if j >= 0:
                g[j].wait()
                issue_out(j)
        for j in range(nwk - (_NB - 1), nwk):
            g[j].wait()
            issue_out(j)
        for j in range(nwk - _NB, nwk):
            o[j].wait()

    return gather4


def _mlp_body(sg8, pg8, sm8, pm8, sidb, pidb,
              w1a, w1b, b1, w2, b2, woh, wog, bo, out):
    hi = jax.lax.Precision.HIGHEST
    soff = sidb[...] & (GRP - 1)
    poff = pidb[...] & (GRP - 1)
    blk = sg8.shape[0]
    jj = lax.broadcasted_iota(jnp.int32, (blk, GRP * D), 1) // D
    ms = (jj == soff).astype(jnp.float32)
    mp = (jj == poff).astype(jnp.float32)
    fr = lax.broadcasted_iota(jnp.int32, (GRP * D, D), 0) % D
    fc = lax.broadcasted_iota(jnp.int32, (GRP * D, D), 1)
    F = (fr == fc).astype(jnp.float32)

    sg = jnp.dot(sg8[...] * ms, F, precision=hi)
    pg = jnp.dot(pg8[...] * mp, F, precision=hi)
    sm = jnp.dot(sm8[...] * ms, F, precision=hi)
    pm = jnp.dot(pm8[...] * mp, F, precision=hi)

    gmf = sg * pg
    h1 = jnp.maximum(jnp.dot(sm, w1a[...], precision=hi)
                     + jnp.dot(pm, w1b[...], precision=hi) + b1[...], 0.0)
    h2 = jnp.maximum(jnp.dot(h1, w2[...], precision=hi) + b2[...], 0.0)
    z = (jnp.sum(h2 * woh[...], axis=1, keepdims=True)
         + jnp.sum(gmf * wog[...], axis=1, keepdims=True)
         + bo[...])
    out[...] = jnp.maximum(z, 0.0)


_BLK = 2048


def _mlp(sg8, pg8, sm8, pm8, sidb, pidb,
         w1a, w1b, b1, w2, b2, woh, wog, bo, interpret=False):
    row = lambda i: (i, 0)
    full = lambda i: (0, 0)
    return pl.pallas_call(
        _mlp_body,
        grid=(B // _BLK,),
        in_specs=[
            pl.BlockSpec((_BLK, GRP * D), row),
            pl.BlockSpec((_BLK, GRP * D), row),
            pl.BlockSpec((_BLK, GRP * D), row),
            pl.BlockSpec((_BLK, GRP * D), row),
            pl.BlockSpec((_BLK, 1), row),
            pl.BlockSpec((_BLK, 1), row),
            pl.BlockSpec((D, 32), full),
            pl.BlockSpec((D, 32), full),
            pl.BlockSpec((1, 32), full),
            pl.BlockSpec((32, D), full),
            pl.BlockSpec((1, D), full),
            pl.BlockSpec((1, D), full),
            pl.BlockSpec((1, D), full),
            pl.BlockSpec((1, 1), full),
        ],
        out_specs=pl.BlockSpec((_BLK, 1), row),
        out_shape=jax.ShapeDtypeStruct((B, 1), jnp.float32),
        interpret=interpret,
    )(sg8, pg8, sm8, pm8, sidb, pidb,
      w1a, w1b, b1, w2, b2, woh, wog, bo)


def kernel(sid, pid, E_sg, E_pg, E_sm, E_pm, W1, b1, W2, b2, Wo, bo):
    sid = sid.astype(jnp.int32)
    pid = pid.astype(jnp.int32)
    sid8 = sid // GRP
    pid8 = pid // GRP
    t_sg = E_sg.reshape(TROWS, GRP * D)
    t_pg = E_pg.reshape(TROWS, GRP * D)
    t_sm = E_sm.reshape(TROWS, GRP * D)
    t_pm = E_pm.reshape(TROWS, GRP * D)
    sg8, pg8, sm8, pm8 = _build_gather4()(sid8, pid8, t_sg, t_pg, t_sm, t_pm)
    w1a = W1[:D]
    w1b = W1[D:]
    woh = Wo[:D].reshape(1, D)
    wog = Wo[D:].reshape(1, D)
    out = _mlp(sg8, pg8, sm8, pm8, sid.reshape(B, 1), pid.reshape(B, 1),
               w1a, w1b, b1.reshape(1, 32), W2,
               b2.reshape(1, D), woh, wog, bo.reshape(1, 1))
    return out.reshape(B)
